# Initial kernel scaffold; baseline (speedup 1.0000x reference)
import jax
import jax.numpy as jnp
from jax.experimental import pallas as pl


def kernel(cls_score, predicted_bbox, bbox_distribution, num_positive_anchors, batch_ids, feat_ids, class_labels, target_boxes):
    raise NotImplementedError("write your pallas kernel here")


# speedup vs baseline: 3.0538x; 3.0538x over previous
import functools

import jax
import numpy as np
import jax.numpy as jnp
from jax import lax
from jax.experimental import pallas as pl
from jax.experimental.pallas import tpu as pltpu
from jax.experimental.pallas import tpu_sc as plsc

_N, _C, _H, _W = 16, 80, 100, 100
_HW = _H * _W
_P = 2048
_REGN = 17


def _softplus(x):
    return jnp.maximum(x, 0.0) + jnp.log1p(jnp.exp(-jnp.abs(x)))


def _dense_body(x_ref, lab_ref, s_ref, wt_ref, pt_ref):
    b = pl.program_id(0)
    x = x_ref[0]
    lab = lab_ref[0]
    sig = jax.nn.sigmoid(x)
    neg = _softplus(x) * sig * sig
    cid = lax.broadcasted_iota(jnp.int32, x.shape, 0)
    sel = cid == lab[None]
    pt = jnp.sum(jnp.where(sel, x, 0.0), axis=0)
    wt = jnp.max(x, axis=0)
    wt_ref[0] = wt
    pt_ref[0] = pt

    @pl.when(b == 0)
    def _():
        s_ref[...] = jnp.zeros((1, 1), jnp.float32)

    s_ref[...] += jnp.sum(neg).reshape(1, 1)


def _dense_pass(cls_score, lab):
    return pl.pallas_call(
        _dense_body,
        grid=(_N,),
        in_specs=[
            pl.BlockSpec((1, _C, _H, _W), lambda b: (b, 0, 0, 0)),
            pl.BlockSpec((1, _H, _W), lambda b: (b, 0, 0)),
        ],
        out_specs=[
            pl.BlockSpec((1, 1), lambda b: (0, 0)),
            pl.BlockSpec((1, _H, _W), lambda b: (b, 0, 0)),
            pl.BlockSpec((1, _H, _W), lambda b: (b, 0, 0)),
        ],
        out_shape=[
            jax.ShapeDtypeStruct((1, 1), jnp.float32),
            jax.ShapeDtypeStruct((_N, _H, _W), jnp.float32),
            jax.ShapeDtypeStruct((_N, _H, _W), jnp.float32),
        ],
        compiler_params=pltpu.CompilerParams(
            dimension_semantics=("arbitrary",)),
    )(cls_score, lab)


@functools.cache
def _sc_gather_kernel():
    info = plsc.get_sparse_core_info()
    nc = info.num_cores
    nw = nc * info.num_subcores
    rp = _P // nw
    nbd = 4 * _REGN

    lanes = 16
    chunk = 128
    rows4 = (rp * 4) // chunk
    rows68 = (rp * nbd) // chunk

    def _fill_indices(idx_v, out_m, width, nrows, divide):
        lane = lax.iota(jnp.int32, lanes)
        for k in range(nrows):
            for c in range(chunk // lanes):
                g = (k * chunk + c * lanes) + lane
                p = divide(g)
                j = g - p * width
                f = plsc.load_gather(idx_v, [p])
                out_m[k, pl.ds(c * lanes, lanes)] = f * width + j

    def body(flat_hbm, pbf_hbm, bdf_hbm, ptf_hbm, wtf_hbm,
             pb_out, bd_out, pt_out, wt_out,
             idx_v, idx4_m, idx68_m, pb_v, bd_v, pt_v, wt_v, sem):
        wid = lax.axis_index("s") * nc + lax.axis_index("c")
        base = wid * rp
        pltpu.sync_copy(flat_hbm.at[pl.ds(base, rp)], idx_v)
        _fill_indices(idx_v, idx4_m, 4, rows4, lambda g: g >> 2)
        _fill_indices(idx_v, idx68_m, nbd, rows68,
                      lambda g: (g * 7711) >> 19)
        copies = [pltpu.async_copy(ptf_hbm.at[idx_v], pt_v, sem),
                  pltpu.async_copy(wtf_hbm.at[idx_v], wt_v, sem)]
        for k in range(rows4):
            copies.append(
                pltpu.async_copy(pbf_hbm.at[idx4_m.at[k]], pb_v.at[k], sem))
        for k in range(rows68):
            copies.append(
                pltpu.async_copy(bdf_hbm.at[idx68_m.at[k]], bd_v.at[k], sem))
        for cp in copies:
            cp.wait()
        pltpu.sync_copy(pb_v, pb_out.at[pl.ds(wid * rows4, rows4)])
        pltpu.sync_copy(bd_v, bd_out.at[pl.ds(wid * rows68, rows68)])
        pltpu.sync_copy(pt_v, pt_out.at[pl.ds(base, rp)])
        pltpu.sync_copy(wt_v, wt_out.at[pl.ds(base, rp)])

    return pl.kernel(
        body,
        out_type=[
            jax.ShapeDtypeStruct((nw * rows4, chunk), jnp.float32),
            jax.ShapeDtypeStruct((nw * rows68, chunk), jnp.float32),
            jax.ShapeDtypeStruct((_P,), jnp.float32),
            jax.ShapeDtypeStruct((_P,), jnp.float32),
        ],
        mesh=plsc.VectorSubcoreMesh(core_axis_name="c",
                                    subcore_axis_name="s"),
        scratch_types=[
            pltpu.VMEM((rp,), jnp.int32),
            pltpu.VMEM((rows4, chunk), jnp.int32),
            pltpu.VMEM((rows68, chunk), jnp.int32),
            pltpu.VMEM((rows4, chunk), jnp.float32),
            pltpu.VMEM((rows68, chunk), jnp.float32),
            pltpu.VMEM((rp,), jnp.float32),
            pltpu.VMEM((rp,), jnp.float32),
            pltpu.SemaphoreType.DMA,
        ],
        compiler_params=pltpu.CompilerParams(use_tc_tiling_on_sc=False,
                                             needs_layout_passes=False),
    )


def _xyxy(b):
    cx, cy, w, h = b[:, 0:1], b[:, 1:2], b[:, 2:3], b[:, 3:4]
    return cx - 0.5 * w, cy - 0.5 * h, cx + 0.5 * w, cy + 0.5 * h


def _sparse_body(pb_ref, tb_ref, bd_ref, lab_ref, pt_ref, wt_ref, wt4_ref,
                 o_ref):
    ax1, ay1, ax2, ay2 = _xyxy(pb_ref[...])
    bx1, by1, bx2, by2 = _xyxy(tb_ref[...])
    iw = jnp.clip(jnp.minimum(ax2, bx2) - jnp.maximum(ax1, bx1), 0.0)
    ih = jnp.clip(jnp.minimum(ay2, by2) - jnp.maximum(ay1, by1), 0.0)
    inter = iw * ih
    union = (ax2 - ax1) * (ay2 - ay1) + (bx2 - bx1) * (by2 - by1) - inter
    iou = inter / jnp.maximum(union, 1e-6)
    ew = jnp.clip(jnp.maximum(ax2, bx2) - jnp.minimum(ax1, bx1), 0.0)
    eh = jnp.clip(jnp.maximum(ay2, by2) - jnp.minimum(ay1, by1), 0.0)
    enclose = ew * eh
    l_giou = 1.0 - (iou - (enclose - union) / jnp.maximum(enclose, 1e-6))

    pt = pt_ref[...]
    q = iou
    sigt = jax.nn.sigmoid(pt)
    sp_t = _softplus(pt)
    d = q - sigt
    corr = (sp_t - pt * q) * d * d - sp_t * sigt * sigt

    bd = bd_ref[...]
    labf = lab_ref[...]
    dl = jnp.floor(labf)
    wl = dl + 1.0 - labf
    wr = labf - dl
    dli = dl.astype(jnp.int32)
    dri = jnp.clip(dli + 1, 0, _REGN - 1)
    m = jnp.max(bd, axis=1, keepdims=True)
    lse = m + jnp.log(jnp.sum(jnp.exp(bd - m), axis=1, keepdims=True))
    ii = lax.broadcasted_iota(jnp.int32, bd.shape, 1)
    pick_l = jnp.sum(jnp.where(ii == dli, bd, 0.0), axis=1, keepdims=True)
    pick_r = jnp.sum(jnp.where(ii == dri, bd, 0.0), axis=1, keepdims=True)
    dfl = (lse - pick_l) * wl + (lse - pick_r) * wr

    wt = wt_ref[...]
    o_ref[...] = jnp.concatenate(
        [jnp.sum(corr).reshape(1, 1),
         jnp.sum(wt).reshape(1, 1),
         jnp.sum(l_giou * wt).reshape(1, 1),
         jnp.sum(dfl * wt4_ref[...]).reshape(1, 1)], axis=1)


def _sparse_pass(pb, tb, bd17, lab17, pt, wt, wt4):
    return pl.pallas_call(
        _sparse_body,
        out_shape=jax.ShapeDtypeStruct((1, 4), jnp.float32),
    )(pb, tb, bd17, lab17, pt, wt, wt4)


def kernel(cls_score, predicted_bbox, bbox_distribution, num_positive_anchors,
           batch_ids, feat_ids, class_labels, target_boxes):
    lab = class_labels.astype(jnp.int32).reshape(_N, _H, _W)
    s_neg, wt_dense, pt_dense = _dense_pass(cls_score, lab)

    flat_pos = (batch_ids.astype(jnp.int32) * _HW
                + feat_ids.astype(jnp.int32))
    pbf = predicted_bbox.reshape(-1)
    bdf = bbox_distribution.reshape(-1)
    pb, bd, ptv, wtv = _sc_gather_kernel()(
        flat_pos, pbf, bdf,
        pt_dense.reshape(_N * _HW), wt_dense.reshape(_N * _HW))

    pb = pb.reshape(_P, 4)
    bd17 = bd.reshape(4 * _P, _REGN)
    lab17 = (target_boxes.reshape(4 * _P, 1) * (_REGN - 1.0))
    o = _sparse_pass(pb, target_boxes, bd17, lab17,
                     ptv.reshape(_P, 1), wtv.reshape(_P, 1),
                     jnp.repeat(wtv, 4).reshape(4 * _P, 1))

    num_pos = jnp.maximum(num_positive_anchors, 1.0)
    qfl_sum = s_neg[0, 0] + o[0, 0]
    wt_sum = o[0, 1]
    loss_qfl = qfl_sum / num_pos
    loss_iou = o[0, 2] / wt_sum
    loss_dfl = (o[0, 3] / 4.0) / wt_sum
    loss = loss_qfl + 2.0 * loss_iou + 0.25 * loss_dfl
    return loss, loss_qfl, loss_iou, loss_dfl


# speedup vs baseline: 3.7616x; 1.2318x over previous
import functools

import jax
import numpy as np
import jax.numpy as jnp
from jax import lax
from jax.experimental import pallas as pl
from jax.experimental.pallas import tpu as pltpu
from jax.experimental.pallas import tpu_sc as plsc

_N, _C, _H, _W = 16, 80, 100, 100
_HW = _H * _W
_P = 2048
_REGN = 17


def _softplus(x):
    return jnp.maximum(x, 0.0) + jnp.log1p(jnp.exp(-jnp.abs(x)))


def _dense_body(x_ref, lab_ref, s_ref, wt_ref, pt_ref):
    b = pl.program_id(0)
    x = x_ref[0]
    lab = lab_ref[0]
    sig = jax.nn.sigmoid(x)
    neg = _softplus(x) * sig * sig
    cid = lax.broadcasted_iota(jnp.int32, x.shape, 1)
    sel = cid == lab[:, None, :]
    pt = jnp.sum(jnp.where(sel, x, 0.0), axis=1)
    wt = jnp.max(x, axis=1)
    wt_ref[0] = wt
    pt_ref[0] = pt

    @pl.when(b == 0)
    def _():
        s_ref[...] = jnp.zeros((1, 1), jnp.float32)

    s_ref[...] += jnp.sum(neg).reshape(1, 1)


def _dense_pass(cls_t, lab):
    return pl.pallas_call(
        _dense_body,
        grid=(_N,),
        in_specs=[
            pl.BlockSpec((1, _H, _C, _W), lambda b: (b, 0, 0, 0)),
            pl.BlockSpec((1, _H, _W), lambda b: (b, 0, 0)),
        ],
        out_specs=[
            pl.BlockSpec((1, 1), lambda b: (0, 0)),
            pl.BlockSpec((1, _H, _W), lambda b: (b, 0, 0)),
            pl.BlockSpec((1, _H, _W), lambda b: (b, 0, 0)),
        ],
        out_shape=[
            jax.ShapeDtypeStruct((1, 1), jnp.float32),
            jax.ShapeDtypeStruct((_N, _H, _W), jnp.float32),
            jax.ShapeDtypeStruct((_N, _H, _W), jnp.float32),
        ],
        compiler_params=pltpu.CompilerParams(
            dimension_semantics=("arbitrary",)),
    )(cls_t, lab)


@functools.cache
def _sc_gather_kernel():
    info = plsc.get_sparse_core_info()
    nc = info.num_cores
    nw = nc * info.num_subcores
    rp = _P // nw
    nbd = 4 * _REGN

    lanes = 16
    chunk = 128
    rows4 = (rp * 4) // chunk
    rows68 = (rp * nbd) // chunk

    def _fill_indices(idx_v, out_m, width, nrows, divide):
        lane = lax.iota(jnp.int32, lanes)
        for k in range(nrows):
            for c in range(chunk // lanes):
                g = (k * chunk + c * lanes) + lane
                p = divide(g)
                j = g - p * width
                f = plsc.load_gather(idx_v, [p])
                out_m[k, pl.ds(c * lanes, lanes)] = f * width + j

    def _fill_bd_indices(bid_v, fid_v, out_m):
        lane = lax.iota(jnp.int32, lanes)
        for k in range(rows68):
            for c in range(chunk // lanes):
                g = (k * chunk + c * lanes) + lane
                p = (g * 7711) >> 19
                j = g - p * nbd
                b16 = plsc.load_gather(bid_v, [p])
                f16 = plsc.load_gather(fid_v, [p])
                h = (f16 * 5243) >> 19
                w = f16 - h * 100
                out_m[k, pl.ds(c * lanes, lanes)] = (
                    ((h * nbd + j) * _N + b16) * 100 + w)

    def body(flat_hbm, bid_hbm, fid_hbm, pbf_hbm, bdf_hbm, ptf_hbm, wtf_hbm,
             pb_out, bd_out, pt_out, wt_out,
             idx_v, bid_v, fid_v, idx4_m, idx68_m, pb_v, bd_v, pt_v, wt_v,
             sem):
        wid = lax.axis_index("s") * nc + lax.axis_index("c")
        base = wid * rp
        pltpu.sync_copy(flat_hbm.at[pl.ds(base, rp)], idx_v)
        pltpu.sync_copy(bid_hbm.at[pl.ds(base, rp)], bid_v)
        pltpu.sync_copy(fid_hbm.at[pl.ds(base, rp)], fid_v)
        _fill_indices(idx_v, idx4_m, 4, rows4, lambda g: g >> 2)
        _fill_bd_indices(bid_v, fid_v, idx68_m)
        copies = [pltpu.async_copy(ptf_hbm.at[idx_v], pt_v, sem),
                  pltpu.async_copy(wtf_hbm.at[idx_v], wt_v, sem)]
        for k in range(rows4):
            copies.append(
                pltpu.async_copy(pbf_hbm.at[idx4_m.at[k]], pb_v.at[k], sem))
        for k in range(rows68):
            copies.append(
                pltpu.async_copy(bdf_hbm.at[idx68_m.at[k]], bd_v.at[k], sem))
        for cp in copies:
            cp.wait()
        pltpu.sync_copy(pb_v, pb_out.at[pl.ds(wid * rows4, rows4)])
        pltpu.sync_copy(bd_v, bd_out.at[pl.ds(wid * rows68, rows68)])
        pltpu.sync_copy(pt_v, pt_out.at[pl.ds(base, rp)])
        pltpu.sync_copy(wt_v, wt_out.at[pl.ds(base, rp)])

    return pl.kernel(
        body,
        out_type=[
            jax.ShapeDtypeStruct((nw * rows4, chunk), jnp.float32),
            jax.ShapeDtypeStruct((nw * rows68, chunk), jnp.float32),
            jax.ShapeDtypeStruct((_P,), jnp.float32),
            jax.ShapeDtypeStruct((_P,), jnp.float32),
        ],
        mesh=plsc.VectorSubcoreMesh(core_axis_name="c",
                                    subcore_axis_name="s"),
        scratch_types=[
            pltpu.VMEM((rp,), jnp.int32),
            pltpu.VMEM((rp,), jnp.int32),
            pltpu.VMEM((rp,), jnp.int32),
            pltpu.VMEM((rows4, chunk), jnp.int32),
            pltpu.VMEM((rows68, chunk), jnp.int32),
            pltpu.VMEM((rows4, chunk), jnp.float32),
            pltpu.VMEM((rows68, chunk), jnp.float32),
            pltpu.VMEM((rp,), jnp.float32),
            pltpu.VMEM((rp,), jnp.float32),
            pltpu.SemaphoreType.DMA,
        ],
        compiler_params=pltpu.CompilerParams(use_tc_tiling_on_sc=False,
                                             needs_layout_passes=False),
    )


def _xyxy(b):
    cx, cy, w, h = b[:, 0:1], b[:, 1:2], b[:, 2:3], b[:, 3:4]
    return cx - 0.5 * w, cy - 0.5 * h, cx + 0.5 * w, cy + 0.5 * h


def _sparse_body(pb_ref, tb_ref, bd_ref, lab_ref, pt_ref, wt_ref, wt4_ref,
                 o_ref):
    ax1, ay1, ax2, ay2 = _xyxy(pb_ref[...])
    bx1, by1, bx2, by2 = _xyxy(tb_ref[...])
    iw = jnp.clip(jnp.minimum(ax2, bx2) - jnp.maximum(ax1, bx1), 0.0)
    ih = jnp.clip(jnp.minimum(ay2, by2) - jnp.maximum(ay1, by1), 0.0)
    inter = iw * ih
    union = (ax2 - ax1) * (ay2 - ay1) + (bx2 - bx1) * (by2 - by1) - inter
    iou = inter / jnp.maximum(union, 1e-6)
    ew = jnp.clip(jnp.maximum(ax2, bx2) - jnp.minimum(ax1, bx1), 0.0)
    eh = jnp.clip(jnp.maximum(ay2, by2) - jnp.minimum(ay1, by1), 0.0)
    enclose = ew * eh
    l_giou = 1.0 - (iou - (enclose - union) / jnp.maximum(enclose, 1e-6))

    pt = pt_ref[...]
    q = iou
    sigt = jax.nn.sigmoid(pt)
    sp_t = _softplus(pt)
    d = q - sigt
    corr = (sp_t - pt * q) * d * d - sp_t * sigt * sigt

    bd = bd_ref[...]
    labf = lab_ref[...]
    dl = jnp.floor(labf)
    wl = dl + 1.0 - labf
    wr = labf - dl
    dli = dl.astype(jnp.int32)
    dri = jnp.clip(dli + 1, 0, _REGN - 1)
    m = jnp.max(bd, axis=1, keepdims=True)
    lse = m + jnp.log(jnp.sum(jnp.exp(bd - m), axis=1, keepdims=True))
    ii = lax.broadcasted_iota(jnp.int32, bd.shape, 1)
    pick_l = jnp.sum(jnp.where(ii == dli, bd, 0.0), axis=1, keepdims=True)
    pick_r = jnp.sum(jnp.where(ii == dri, bd, 0.0), axis=1, keepdims=True)
    dfl = (lse - pick_l) * wl + (lse - pick_r) * wr

    wt = wt_ref[...]
    o_ref[...] = jnp.concatenate(
        [jnp.sum(corr).reshape(1, 1),
         jnp.sum(wt).reshape(1, 1),
         jnp.sum(l_giou * wt).reshape(1, 1),
         jnp.sum(dfl * wt4_ref[...]).reshape(1, 1)], axis=1)


def _sparse_pass(pb, tb, bd17, lab17, pt, wt, wt4):
    return pl.pallas_call(
        _sparse_body,
        out_shape=jax.ShapeDtypeStruct((1, 4), jnp.float32),
    )(pb, tb, bd17, lab17, pt, wt, wt4)


def kernel(cls_score, predicted_bbox, bbox_distribution, num_positive_anchors,
           batch_ids, feat_ids, class_labels, target_boxes):
    lab = class_labels.astype(jnp.int32).reshape(_N, _H, _W)
    cls_t = jnp.transpose(cls_score, (0, 2, 1, 3))
    s_neg, wt_dense, pt_dense = _dense_pass(cls_t, lab)

    flat_pos = (batch_ids.astype(jnp.int32) * _HW
                + feat_ids.astype(jnp.int32))
    pbf = predicted_bbox.reshape(-1)
    bdf = jnp.transpose(bbox_distribution, (1, 3, 0, 2)).reshape(-1)
    pb, bd, ptv, wtv = _sc_gather_kernel()(
        flat_pos, batch_ids.astype(jnp.int32), feat_ids.astype(jnp.int32),
        pbf, bdf,
        pt_dense.reshape(_N * _HW), wt_dense.reshape(_N * _HW))

    pb = pb.reshape(_P, 4)
    bd17 = bd.reshape(4 * _P, _REGN)
    lab17 = (target_boxes.reshape(4 * _P, 1) * (_REGN - 1.0))
    o = _sparse_pass(pb, target_boxes, bd17, lab17,
                     ptv.reshape(_P, 1), wtv.reshape(_P, 1),
                     jnp.repeat(wtv, 4).reshape(4 * _P, 1))

    num_pos = jnp.maximum(num_positive_anchors, 1.0)
    qfl_sum = s_neg[0, 0] + o[0, 0]
    wt_sum = o[0, 1]
    loss_qfl = qfl_sum / num_pos
    loss_iou = o[0, 2] / wt_sum
    loss_dfl = (o[0, 3] / 4.0) / wt_sum
    loss = loss_qfl + 2.0 * loss_iou + 0.25 * loss_dfl
    return loss, loss_qfl, loss_iou, loss_dfl


# speedup vs baseline: 4.7217x; 1.2552x over previous
import functools

import jax
import numpy as np
import jax.numpy as jnp
from jax import lax
from jax.experimental import pallas as pl
from jax.experimental.pallas import tpu as pltpu
from jax.experimental.pallas import tpu_sc as plsc

_N, _C, _H, _W = 16, 80, 100, 100
_HW = _H * _W
_P = 2048
_REGN = 17


def _softplus(x):
    return jnp.maximum(x, 0.0) + jnp.log1p(jnp.exp(-jnp.abs(x)))


def _sc_mesh_info():
    info = plsc.get_sparse_core_info()
    nc = info.num_cores
    return nc, nc * info.num_subcores


_LANES = 16
_CHUNK = 128


@functools.cache
def _sc_scatter_kernel():
    nc, nw = _sc_mesh_info()
    rp = _P // nw
    rows4 = (rp * 4) // _CHUNK

    def body(flat_hbm, tbl_hbm, tbd_out, idx_v, scat_m, tb_m, sem):
        wid = lax.axis_index("s") * nc + lax.axis_index("c")
        base = wid * rp
        pltpu.sync_copy(flat_hbm.at[pl.ds(base, rp)], idx_v)
        lane = lax.iota(jnp.int32, _LANES)
        for k in range(rows4):
            pltpu.sync_copy(tbl_hbm.at[pl.ds(base * 4 + k * _CHUNK, _CHUNK)],
                            tb_m.at[k])
            for c in range(_CHUNK // _LANES):
                g = (k * _CHUNK + c * _LANES) + lane
                p = g >> 2
                comp = g - (p << 2)
                f = plsc.load_gather(idx_v, [p])
                bh = (f * 5243) >> 19
                w = f - bh * 100
                scat_m[k, pl.ds(c * _LANES, _LANES)] = (
                    bh * 400 + comp * 100 + w)
        copies = [pltpu.async_copy(tb_m.at[k], tbd_out.at[scat_m.at[k]], sem)
                  for k in range(rows4)]
        for cp in copies:
            cp.wait()

    return pl.kernel(
        body,
        out_type=jax.ShapeDtypeStruct((_N * _HW * 4,), jnp.float32),
        mesh=plsc.VectorSubcoreMesh(core_axis_name="c",
                                    subcore_axis_name="s"),
        scratch_types=[
            pltpu.VMEM((_P // 32,), jnp.int32),
            pltpu.VMEM(((_P // 32) * 4 // _CHUNK, _CHUNK), jnp.int32),
            pltpu.VMEM(((_P // 32) * 4 // _CHUNK, _CHUNK), jnp.float32),
            pltpu.SemaphoreType.DMA,
        ],
        compiler_params=pltpu.CompilerParams(use_tc_tiling_on_sc=False,
                                             needs_layout_passes=False),
    )


def _box_terms(b4):
    cx, cy, w, h = b4[:, 0, :], b4[:, 1, :], b4[:, 2, :], b4[:, 3, :]
    return cx - 0.5 * w, cy - 0.5 * h, cx + 0.5 * w, cy + 0.5 * h


def _dense_body(x_ref, lab_ref, pb_ref, tb_ref, s_ref, wt_ref):
    b = pl.program_id(0)
    x = x_ref[0]
    lab = lab_ref[0]
    sig = jax.nn.sigmoid(x)
    neg = _softplus(x) * sig * sig
    cid = lax.broadcasted_iota(jnp.int32, x.shape, 1)
    sel = cid == lab[:, None, :]
    pt = jnp.sum(jnp.where(sel, x, 0.0), axis=1)
    wt = jnp.max(x, axis=1)
    wt_ref[0] = wt

    pos = lab < _C
    ax1, ay1, ax2, ay2 = _box_terms(pb_ref[0])
    bx1, by1, bx2, by2 = _box_terms(tb_ref[0])
    iw = jnp.clip(jnp.minimum(ax2, bx2) - jnp.maximum(ax1, bx1), 0.0)
    ih = jnp.clip(jnp.minimum(ay2, by2) - jnp.maximum(ay1, by1), 0.0)
    inter = iw * ih
    union = (ax2 - ax1) * (ay2 - ay1) + (bx2 - bx1) * (by2 - by1) - inter
    iou = inter / jnp.maximum(union, 1e-6)
    ew = jnp.clip(jnp.maximum(ax2, bx2) - jnp.minimum(ax1, bx1), 0.0)
    eh = jnp.clip(jnp.maximum(ay2, by2) - jnp.minimum(ay1, by1), 0.0)
    enclose = ew * eh
    l_giou = 1.0 - (iou - (enclose - union) / jnp.maximum(enclose, 1e-6))

    sigt = jax.nn.sigmoid(pt)
    sp_t = _softplus(pt)
    d = iou - sigt
    corr = (sp_t - pt * iou) * d * d - sp_t * sigt * sigt

    wtp = jnp.where(pos, wt, 0.0)
    sums = jnp.concatenate(
        [jnp.sum(neg).reshape(1, 1),
         jnp.sum(jnp.where(pos, corr, 0.0)).reshape(1, 1),
         jnp.sum(wtp).reshape(1, 1),
         jnp.sum(jnp.where(pos, l_giou * wt, 0.0)).reshape(1, 1)], axis=1)

    @pl.when(b == 0)
    def _():
        s_ref[...] = jnp.zeros((1, 4), jnp.float32)

    s_ref[...] += sums


def _dense_pass(cls_t, lab, pb_t, tb_t):
    return pl.pallas_call(
        _dense_body,
        grid=(_N,),
        in_specs=[
            pl.BlockSpec((1, _H, _C, _W), lambda b: (b, 0, 0, 0)),
            pl.BlockSpec((1, _H, _W), lambda b: (b, 0, 0)),
            pl.BlockSpec((1, _H, 4, _W), lambda b: (b, 0, 0, 0)),
            pl.BlockSpec((1, _H, 4, _W), lambda b: (b, 0, 0, 0)),
        ],
        out_specs=[
            pl.BlockSpec((1, 4), lambda b: (0, 0)),
            pl.BlockSpec((1, _H, _W), lambda b: (b, 0, 0)),
        ],
        out_shape=[
            jax.ShapeDtypeStruct((1, 4), jnp.float32),
            jax.ShapeDtypeStruct((_N, _H, _W), jnp.float32),
        ],
        compiler_params=pltpu.CompilerParams(
            dimension_semantics=("arbitrary",)),
    )(cls_t, lab, pb_t, tb_t)


@functools.cache
def _sc_gather_kernel():
    nc, nw = _sc_mesh_info()
    rp = _P // nw
    nbd = 4 * _REGN
    rows68 = (rp * nbd) // _CHUNK

    def _fill_bd_indices(bid_v, fid_v, out_m):
        lane = lax.iota(jnp.int32, _LANES)
        for k in range(rows68):
            for c in range(_CHUNK // _LANES):
                g = (k * _CHUNK + c * _LANES) + lane
                p = (g * 7711) >> 19
                j = g - p * nbd
                b16 = plsc.load_gather(bid_v, [p])
                f16 = plsc.load_gather(fid_v, [p])
                h = (f16 * 5243) >> 19
                w = f16 - h * 100
                out_m[k, pl.ds(c * _LANES, _LANES)] = (
                    ((h * nbd + j) * _N + b16) * 100 + w)

    def body(flat_hbm, bid_hbm, fid_hbm, bdf_hbm, wtf_hbm,
             bd_out, wt_out,
             idx_v, bid_v, fid_v, idx68_m, bd_v, wt_v, sem):
        wid = lax.axis_index("s") * nc + lax.axis_index("c")
        base = wid * rp
        pltpu.sync_copy(flat_hbm.at[pl.ds(base, rp)], idx_v)
        pltpu.sync_copy(bid_hbm.at[pl.ds(base, rp)], bid_v)
        pltpu.sync_copy(fid_hbm.at[pl.ds(base, rp)], fid_v)
        _fill_bd_indices(bid_v, fid_v, idx68_m)
        copies = [pltpu.async_copy(wtf_hbm.at[idx_v], wt_v, sem)]
        for k in range(rows68):
            copies.append(
                pltpu.async_copy(bdf_hbm.at[idx68_m.at[k]], bd_v.at[k], sem))
        for cp in copies:
            cp.wait()
        pltpu.sync_copy(bd_v, bd_out.at[pl.ds(wid * rows68, rows68)])
        pltpu.sync_copy(wt_v, wt_out.at[pl.ds(base, rp)])

    return pl.kernel(
        body,
        out_type=[
            jax.ShapeDtypeStruct((nw * rows68, _CHUNK), jnp.float32),
            jax.ShapeDtypeStruct((_P,), jnp.float32),
        ],
        mesh=plsc.VectorSubcoreMesh(core_axis_name="c",
                                    subcore_axis_name="s"),
        scratch_types=[
            pltpu.VMEM((rp,), jnp.int32),
            pltpu.VMEM((rp,), jnp.int32),
            pltpu.VMEM((rp,), jnp.int32),
            pltpu.VMEM((rows68, _CHUNK), jnp.int32),
            pltpu.VMEM((rows68, _CHUNK), jnp.float32),
            pltpu.VMEM((rp,), jnp.float32),
            pltpu.SemaphoreType.DMA,
        ],
        compiler_params=pltpu.CompilerParams(use_tc_tiling_on_sc=False,
                                             needs_layout_passes=False),
    )


def _dfl_body(bd_ref, lab_ref, wt4_ref, o_ref):
    bd = bd_ref[...]
    labf = lab_ref[...]
    dl = jnp.floor(labf)
    wl = dl + 1.0 - labf
    wr = labf - dl
    dli = dl.astype(jnp.int32)
    dri = jnp.clip(dli + 1, 0, _REGN - 1)
    m = jnp.max(bd, axis=1, keepdims=True)
    lse = m + jnp.log(jnp.sum(jnp.exp(bd - m), axis=1, keepdims=True))
    ii = lax.broadcasted_iota(jnp.int32, bd.shape, 1)
    pick_l = jnp.sum(jnp.where(ii == dli, bd, 0.0), axis=1, keepdims=True)
    pick_r = jnp.sum(jnp.where(ii == dri, bd, 0.0), axis=1, keepdims=True)
    dfl = (lse - pick_l) * wl + (lse - pick_r) * wr
    o_ref[...] = jnp.sum(dfl * wt4_ref[...]).reshape(1, 1)


def _dfl_pass(bd17, lab17, wt4):
    return pl.pallas_call(
        _dfl_body,
        out_shape=jax.ShapeDtypeStruct((1, 1), jnp.float32),
    )(bd17, lab17, wt4)


def kernel(cls_score, predicted_bbox, bbox_distribution, num_positive_anchors,
           batch_ids, feat_ids, class_labels, target_boxes):
    lab = class_labels.astype(jnp.int32).reshape(_N, _H, _W)
    cls_t = jnp.transpose(cls_score, (0, 2, 1, 3))
    pb_t = jnp.transpose(predicted_bbox, (0, 1, 3, 2))

    flat_pos = (batch_ids.astype(jnp.int32) * _HW
                + feat_ids.astype(jnp.int32))
    tbd = _sc_scatter_kernel()(flat_pos, target_boxes.reshape(-1))
    tb_t = tbd.reshape(_N, _H, 4, _W)

    s4, wt_dense = _dense_pass(cls_t, lab, pb_t, tb_t)

    bdf = jnp.transpose(bbox_distribution, (1, 3, 0, 2)).reshape(-1)
    bd, wtv = _sc_gather_kernel()(
        flat_pos, batch_ids.astype(jnp.int32), feat_ids.astype(jnp.int32),
        bdf, wt_dense.reshape(_N * _HW))

    bd17 = bd.reshape(4 * _P, _REGN)
    lab17 = (target_boxes.reshape(4 * _P, 1) * (_REGN - 1.0))
    dflw = _dfl_pass(bd17, lab17, jnp.repeat(wtv, 4).reshape(4 * _P, 1))

    num_pos = jnp.maximum(num_positive_anchors, 1.0)
    qfl_sum = s4[0, 0] + s4[0, 1]
    wt_sum = s4[0, 2]
    loss_qfl = qfl_sum / num_pos
    loss_iou = s4[0, 3] / wt_sum
    loss_dfl = (dflw[0, 0] / 4.0) / wt_sum
    loss = loss_qfl + 2.0 * loss_iou + 0.25 * loss_dfl
    return loss, loss_qfl, loss_iou, loss_dfl


# speedup vs baseline: 5.5513x; 1.1757x over previous
import functools

import jax
import numpy as np
import jax.numpy as jnp
from jax import lax
from jax.experimental import pallas as pl
from jax.experimental.pallas import tpu as pltpu
from jax.experimental.pallas import tpu_sc as plsc

_N, _C, _H, _W = 16, 80, 100, 100
_HW = _H * _W
_P = 2048
_REGN = 17


def _softplus(x):
    return jnp.maximum(x, 0.0) + jnp.log1p(jnp.exp(-jnp.abs(x)))


def _sc_mesh_info():
    info = plsc.get_sparse_core_info()
    nc = info.num_cores
    return nc, nc * info.num_subcores


_LANES = 16
_CHUNK = 128


@functools.cache
def _sc_scatter_kernel():
    nc, nw = _sc_mesh_info()
    rp = _P // nw
    rows4 = (rp * 4) // _CHUNK

    def body(flat_hbm, tbl_hbm, tbd_out, idx_v, scat_m, tb_m, sem):
        wid = lax.axis_index("s") * nc + lax.axis_index("c")
        base = wid * rp
        pltpu.sync_copy(flat_hbm.at[pl.ds(base, rp)], idx_v)
        lane = lax.iota(jnp.int32, _LANES)
        for k in range(rows4):
            pltpu.sync_copy(tbl_hbm.at[pl.ds(base * 4 + k * _CHUNK, _CHUNK)],
                            tb_m.at[k])
            for c in range(_CHUNK // _LANES):
                g = (k * _CHUNK + c * _LANES) + lane
                p = g >> 2
                comp = g - (p << 2)
                f = plsc.load_gather(idx_v, [p])
                bh = (f * 5243) >> 19
                w = f - bh * 100
                scat_m[k, pl.ds(c * _LANES, _LANES)] = (
                    bh * 400 + comp * 100 + w)
        copies = [pltpu.async_copy(tb_m.at[k], tbd_out.at[scat_m.at[k]], sem)
                  for k in range(rows4)]
        for cp in copies:
            cp.wait()

    return pl.kernel(
        body,
        out_type=jax.ShapeDtypeStruct((_N * _HW * 4,), jnp.float32),
        mesh=plsc.VectorSubcoreMesh(core_axis_name="c",
                                    subcore_axis_name="s"),
        scratch_types=[
            pltpu.VMEM((_P // 32,), jnp.int32),
            pltpu.VMEM(((_P // 32) * 4 // _CHUNK, _CHUNK), jnp.int32),
            pltpu.VMEM(((_P // 32) * 4 // _CHUNK, _CHUNK), jnp.float32),
            pltpu.SemaphoreType.DMA,
        ],
        compiler_params=pltpu.CompilerParams(use_tc_tiling_on_sc=False,
                                             needs_layout_passes=False),
    )


def _box_terms(b4):
    cx, cy, w, h = b4[:, 0, :], b4[:, 1, :], b4[:, 2, :], b4[:, 3, :]
    return cx - 0.5 * w, cy - 0.5 * h, cx + 0.5 * w, cy + 0.5 * h


def _dense_body(x_ref, lab_ref, pb_ref, tb_ref, bd_ref, s_ref, wt_ref,
                bdp_ref):
    h = pl.program_id(0)
    x = x_ref[:, 0]
    lab = lab_ref[:, 0, 0]
    sig = jax.nn.sigmoid(x)
    neg = _softplus(x) * sig * sig
    cid = lax.broadcasted_iota(jnp.int32, x.shape, 1)
    sel = cid == lab[:, None, :]
    pt = jnp.sum(jnp.where(sel, x, 0.0), axis=1)
    wt = jnp.max(x, axis=1)
    wt_ref[:, 0, 0] = wt

    bdp_ref[0, :, :, pl.ds(0, _W)] = bd_ref[0]

    pos = lab < _C
    ax1, ay1, ax2, ay2 = _box_terms(pb_ref[:, 0])
    bx1, by1, bx2, by2 = _box_terms(tb_ref[:, 0])
    iw = jnp.clip(jnp.minimum(ax2, bx2) - jnp.maximum(ax1, bx1), 0.0)
    ih = jnp.clip(jnp.minimum(ay2, by2) - jnp.maximum(ay1, by1), 0.0)
    inter = iw * ih
    union = (ax2 - ax1) * (ay2 - ay1) + (bx2 - bx1) * (by2 - by1) - inter
    iou = inter / jnp.maximum(union, 1e-6)
    ew = jnp.clip(jnp.maximum(ax2, bx2) - jnp.minimum(ax1, bx1), 0.0)
    eh = jnp.clip(jnp.maximum(ay2, by2) - jnp.minimum(ay1, by1), 0.0)
    enclose = ew * eh
    l_giou = 1.0 - (iou - (enclose - union) / jnp.maximum(enclose, 1e-6))

    sigt = jax.nn.sigmoid(pt)
    sp_t = _softplus(pt)
    d = iou - sigt
    corr = (sp_t - pt * iou) * d * d - sp_t * sigt * sigt

    wtp = jnp.where(pos, wt, 0.0)
    sums = jnp.concatenate(
        [jnp.sum(neg).reshape(1, 1),
         jnp.sum(jnp.where(pos, corr, 0.0)).reshape(1, 1),
         jnp.sum(wtp).reshape(1, 1),
         jnp.sum(jnp.where(pos, l_giou * wt, 0.0)).reshape(1, 1)], axis=1)

    @pl.when(h == 0)
    def _():
        s_ref[...] = jnp.zeros((1, 4), jnp.float32)

    s_ref[...] += sums


def _dense_pass(cls_t, lab, pb_t, tb_t, bd_t):
    return pl.pallas_call(
        _dense_body,
        grid=(_H,),
        in_specs=[
            pl.BlockSpec((_N, 1, _C, _W), lambda h: (0, h, 0, 0)),
            pl.BlockSpec((_N, 1, 1, _W), lambda h: (0, h, 0, 0)),
            pl.BlockSpec((_N, 1, 4, _W), lambda h: (0, h, 0, 0)),
            pl.BlockSpec((_N, 1, 4, _W), lambda h: (0, h, 0, 0)),
            pl.BlockSpec((1, 4 * _REGN, _N, _W), lambda h: (h, 0, 0, 0)),
        ],
        out_specs=[
            pl.BlockSpec((1, 4), lambda h: (0, 0)),
            pl.BlockSpec((_N, 1, 1, _W), lambda h: (0, h, 0, 0)),
            pl.BlockSpec((1, 4 * _REGN, _N, 128), lambda h: (h, 0, 0, 0)),
        ],
        out_shape=[
            jax.ShapeDtypeStruct((1, 4), jnp.float32),
            jax.ShapeDtypeStruct((_N, _H, 1, _W), jnp.float32),
            jax.ShapeDtypeStruct((_H, 4 * _REGN, _N, 128), jnp.float32),
        ],
        compiler_params=pltpu.CompilerParams(
            dimension_semantics=("arbitrary",)),
    )(cls_t, lab, pb_t, tb_t, bd_t)


@functools.cache
def _sc_gather_kernel():
    nc, nw = _sc_mesh_info()
    rp = _P // nw
    nbd = 4 * _REGN
    rows68 = (rp * nbd) // _CHUNK

    def _fill_bd_indices(bid_v, fid_v, out_m):
        lane = lax.iota(jnp.int32, _LANES)
        for k in range(rows68):
            for c in range(_CHUNK // _LANES):
                g = (k * _CHUNK + c * _LANES) + lane
                p = (g * 7711) >> 19
                j = g - p * nbd
                b16 = plsc.load_gather(bid_v, [p])
                f16 = plsc.load_gather(fid_v, [p])
                h = (f16 * 5243) >> 19
                w = f16 - h * 100
                out_m[k, pl.ds(c * _LANES, _LANES)] = (
                    ((h * nbd + j) * _N + b16) * 128 + w)

    def body(flat_hbm, bid_hbm, fid_hbm, bdf_hbm, wtf_hbm,
             bd_out, wt_out,
             idx_v, bid_v, fid_v, idx68_m, bd_v, wt_v, sem):
        wid = lax.axis_index("s") * nc + lax.axis_index("c")
        base = wid * rp
        pltpu.sync_copy(flat_hbm.at[pl.ds(base, rp)], idx_v)
        pltpu.sync_copy(bid_hbm.at[pl.ds(base, rp)], bid_v)
        pltpu.sync_copy(fid_hbm.at[pl.ds(base, rp)], fid_v)
        _fill_bd_indices(bid_v, fid_v, idx68_m)
        copies = [pltpu.async_copy(wtf_hbm.at[idx_v], wt_v, sem)]
        for k in range(rows68):
            copies.append(
                pltpu.async_copy(bdf_hbm.at[idx68_m.at[k]], bd_v.at[k], sem))
        for cp in copies:
            cp.wait()
        pltpu.sync_copy(bd_v, bd_out.at[pl.ds(wid * rows68, rows68)])
        pltpu.sync_copy(wt_v, wt_out.at[pl.ds(base, rp)])

    return pl.kernel(
        body,
        out_type=[
            jax.ShapeDtypeStruct((nw * rows68, _CHUNK), jnp.float32),
            jax.ShapeDtypeStruct((_P,), jnp.float32),
        ],
        mesh=plsc.VectorSubcoreMesh(core_axis_name="c",
                                    subcore_axis_name="s"),
        scratch_types=[
            pltpu.VMEM((rp,), jnp.int32),
            pltpu.VMEM((rp,), jnp.int32),
            pltpu.VMEM((rp,), jnp.int32),
            pltpu.VMEM((rows68, _CHUNK), jnp.int32),
            pltpu.VMEM((rows68, _CHUNK), jnp.float32),
            pltpu.VMEM((rp,), jnp.float32),
            pltpu.SemaphoreType.DMA,
        ],
        compiler_params=pltpu.CompilerParams(use_tc_tiling_on_sc=False,
                                             needs_layout_passes=False),
    )


def _dfl_body(bd_ref, lab_ref, wt4_ref, o_ref):
    bd = bd_ref[...]
    labf = lab_ref[...]
    dl = jnp.floor(labf)
    wl = dl + 1.0 - labf
    wr = labf - dl
    dli = dl.astype(jnp.int32)
    dri = jnp.clip(dli + 1, 0, _REGN - 1)
    m = jnp.max(bd, axis=1, keepdims=True)
    lse = m + jnp.log(jnp.sum(jnp.exp(bd - m), axis=1, keepdims=True))
    ii = lax.broadcasted_iota(jnp.int32, bd.shape, 1)
    pick_l = jnp.sum(jnp.where(ii == dli, bd, 0.0), axis=1, keepdims=True)
    pick_r = jnp.sum(jnp.where(ii == dri, bd, 0.0), axis=1, keepdims=True)
    dfl = (lse - pick_l) * wl + (lse - pick_r) * wr
    o_ref[...] = jnp.sum(dfl * wt4_ref[...]).reshape(1, 1)


def _dfl_pass(bd17, lab17, wt4):
    return pl.pallas_call(
        _dfl_body,
        out_shape=jax.ShapeDtypeStruct((1, 1), jnp.float32),
    )(bd17, lab17, wt4)


def kernel(cls_score, predicted_bbox, bbox_distribution, num_positive_anchors,
           batch_ids, feat_ids, class_labels, target_boxes):
    lab = class_labels.astype(jnp.int32).reshape(_N, _H, 1, _W)
    cls_t = jnp.transpose(cls_score, (0, 2, 1, 3))
    pb_t = jnp.transpose(predicted_bbox, (0, 1, 3, 2))
    bd_t = jnp.transpose(bbox_distribution, (1, 3, 0, 2))

    flat_pos = (batch_ids.astype(jnp.int32) * _HW
                + feat_ids.astype(jnp.int32))
    tbd = _sc_scatter_kernel()(flat_pos, target_boxes.reshape(-1))
    tb_t = tbd.reshape(_N, _H, 4, _W)

    s4, wt_dense, bd_pad = _dense_pass(cls_t, lab, pb_t, tb_t, bd_t)

    bdf = bd_pad.reshape(-1)
    bd, wtv = _sc_gather_kernel()(
        flat_pos, batch_ids.astype(jnp.int32), feat_ids.astype(jnp.int32),
        bdf, wt_dense.reshape(_N * _HW))

    bd17 = bd.reshape(4 * _P, _REGN)
    lab17 = (target_boxes.reshape(4 * _P, 1) * (_REGN - 1.0))
    dflw = _dfl_pass(bd17, lab17, jnp.repeat(wtv, 4).reshape(4 * _P, 1))

    num_pos = jnp.maximum(num_positive_anchors, 1.0)
    qfl_sum = s4[0, 0] + s4[0, 1]
    wt_sum = s4[0, 2]
    loss_qfl = qfl_sum / num_pos
    loss_iou = s4[0, 3] / wt_sum
    loss_dfl = (dflw[0, 0] / 4.0) / wt_sum
    loss = loss_qfl + 2.0 * loss_iou + 0.25 * loss_dfl
    return loss, loss_qfl, loss_iou, loss_dfl


# speedup vs baseline: 6.0884x; 1.0967x over previous
import functools

import jax
import numpy as np
import jax.numpy as jnp
from jax import lax
from jax.experimental import pallas as pl
from jax.experimental.pallas import tpu as pltpu
from jax.experimental.pallas import tpu_sc as plsc

_N, _C, _H, _W = 16, 80, 100, 100
_HW = _H * _W
_P = 2048
_REGN = 17


def _softplus(x):
    return jnp.maximum(x, 0.0) + jnp.log1p(jnp.exp(-jnp.abs(x)))


def _sc_mesh_info():
    info = plsc.get_sparse_core_info()
    nc = info.num_cores
    return nc, nc * info.num_subcores


_LANES = 16
_CHUNK = 128


@functools.cache
def _sc_scatter_kernel():
    nc, nw = _sc_mesh_info()
    rp = _P // nw
    rows4 = (rp * 4) // _CHUNK

    def body(flat_hbm, tbl_hbm, tbd_out, idx_v, scat_m, tb_m, sem):
        wid = lax.axis_index("s") * nc + lax.axis_index("c")
        base = wid * rp
        pltpu.sync_copy(flat_hbm.at[pl.ds(base, rp)], idx_v)
        lane = lax.iota(jnp.int32, _LANES)
        for k in range(rows4):
            pltpu.sync_copy(tbl_hbm.at[pl.ds(base * 4 + k * _CHUNK, _CHUNK)],
                            tb_m.at[k])
            for c in range(_CHUNK // _LANES):
                g = (k * _CHUNK + c * _LANES) + lane
                p = g >> 2
                comp = g - (p << 2)
                f = plsc.load_gather(idx_v, [p])
                bh = (f * 5243) >> 19
                w = f - bh * 100
                scat_m[k, pl.ds(c * _LANES, _LANES)] = (
                    bh * 1024 + comp * 128 + w)
        copies = [pltpu.async_copy(tb_m.at[k], tbd_out.at[scat_m.at[k]], sem)
                  for k in range(rows4)]
        for cp in copies:
            cp.wait()

    return pl.kernel(
        body,
        out_type=jax.ShapeDtypeStruct((_N * _H * 8 * 128,), jnp.float32),
        mesh=plsc.VectorSubcoreMesh(core_axis_name="c",
                                    subcore_axis_name="s"),
        scratch_types=[
            pltpu.VMEM((_P // 32,), jnp.int32),
            pltpu.VMEM(((_P // 32) * 4 // _CHUNK, _CHUNK), jnp.int32),
            pltpu.VMEM(((_P // 32) * 4 // _CHUNK, _CHUNK), jnp.float32),
            pltpu.SemaphoreType.DMA,
        ],
        compiler_params=pltpu.CompilerParams(use_tc_tiling_on_sc=False,
                                             needs_layout_passes=False),
    )


def _box_terms(b4):
    cx, cy, w, h = b4[:, 0, :], b4[:, 1, :], b4[:, 2, :], b4[:, 3, :]
    return cx - 0.5 * w, cy - 0.5 * h, cx + 0.5 * w, cy + 0.5 * h


def _dense_body(x_ref, lab_ref, pb_ref, tb_ref, bd_ref, s_ref, wt_ref,
                bdp_ref):
    h = pl.program_id(0)
    x = x_ref[:, 0]
    lab = lab_ref[:, 0, 0]
    t = jnp.exp(-jnp.abs(x))
    den = 1.0 / (1.0 + t)
    sig = jnp.where(x >= 0.0, den, t * den)
    neg = (jnp.maximum(x, 0.0) + jnp.log1p(t)) * sig * sig
    cid = lax.broadcasted_iota(jnp.int32, x.shape, 1)
    sel = cid == lab[:, None, :]
    pt = jnp.sum(jnp.where(sel, x, 0.0), axis=1)
    wt = jnp.max(x, axis=1)
    wt_ref[:, 0, 0] = wt

    bdp_ref[0, :, :, pl.ds(0, _W)] = bd_ref[0]

    pos = lab < _C
    ax1, ay1, ax2, ay2 = _box_terms(pb_ref[:, 0])
    bx1, by1, bx2, by2 = _box_terms(tb_ref[:, 0, 0:4, 0:_W])
    iw = jnp.clip(jnp.minimum(ax2, bx2) - jnp.maximum(ax1, bx1), 0.0)
    ih = jnp.clip(jnp.minimum(ay2, by2) - jnp.maximum(ay1, by1), 0.0)
    inter = iw * ih
    union = (ax2 - ax1) * (ay2 - ay1) + (bx2 - bx1) * (by2 - by1) - inter
    iou = inter / jnp.maximum(union, 1e-6)
    ew = jnp.clip(jnp.maximum(ax2, bx2) - jnp.minimum(ax1, bx1), 0.0)
    eh = jnp.clip(jnp.maximum(ay2, by2) - jnp.minimum(ay1, by1), 0.0)
    enclose = ew * eh
    l_giou = 1.0 - (iou - (enclose - union) / jnp.maximum(enclose, 1e-6))

    sigt = jax.nn.sigmoid(pt)
    sp_t = _softplus(pt)
    d = iou - sigt
    corr = (sp_t - pt * iou) * d * d - sp_t * sigt * sigt

    wtp = jnp.where(pos, wt, 0.0)
    sums = jnp.concatenate(
        [jnp.sum(neg).reshape(1, 1),
         jnp.sum(jnp.where(pos, corr, 0.0)).reshape(1, 1),
         jnp.sum(wtp).reshape(1, 1),
         jnp.sum(jnp.where(pos, l_giou * wt, 0.0)).reshape(1, 1)], axis=1)

    @pl.when(h == 0)
    def _():
        s_ref[...] = jnp.zeros((1, 4), jnp.float32)

    s_ref[...] += sums


def _dense_pass(cls_t, lab, pb_t, tb_t, bd_t):
    return pl.pallas_call(
        _dense_body,
        grid=(_H,),
        in_specs=[
            pl.BlockSpec((_N, 1, _C, _W), lambda h: (0, h, 0, 0)),
            pl.BlockSpec((_N, 1, 1, _W), lambda h: (0, h, 0, 0)),
            pl.BlockSpec((_N, 1, 4, _W), lambda h: (0, h, 0, 0)),
            pl.BlockSpec((_N, 1, 8, 128), lambda h: (0, h, 0, 0)),
            pl.BlockSpec((1, 4 * _REGN, _N, _W), lambda h: (h, 0, 0, 0)),
        ],
        out_specs=[
            pl.BlockSpec((1, 4), lambda h: (0, 0)),
            pl.BlockSpec((_N, 1, 1, _W), lambda h: (0, h, 0, 0)),
            pl.BlockSpec((1, 4 * _REGN, _N, 128), lambda h: (h, 0, 0, 0)),
        ],
        out_shape=[
            jax.ShapeDtypeStruct((1, 4), jnp.float32),
            jax.ShapeDtypeStruct((_N, _H, 1, _W), jnp.float32),
            jax.ShapeDtypeStruct((_H, 4 * _REGN, _N, 128), jnp.float32),
        ],
        compiler_params=pltpu.CompilerParams(
            dimension_semantics=("arbitrary",)),
    )(cls_t, lab, pb_t, tb_t, bd_t)


@functools.cache
def _sc_gather_kernel():
    nc, nw = _sc_mesh_info()
    rp = _P // nw
    nbd = 4 * _REGN
    rows68 = (rp * nbd) // _CHUNK

    def _fill_bd_indices(bid_v, fid_v, out_m):
        lane = lax.iota(jnp.int32, _LANES)
        for k in range(rows68):
            for c in range(_CHUNK // _LANES):
                g = (k * _CHUNK + c * _LANES) + lane
                p = (g * 7711) >> 19
                j = g - p * nbd
                b16 = plsc.load_gather(bid_v, [p])
                f16 = plsc.load_gather(fid_v, [p])
                h = (f16 * 5243) >> 19
                w = f16 - h * 100
                out_m[k, pl.ds(c * _LANES, _LANES)] = (
                    ((h * nbd + j) * _N + b16) * 128 + w)

    def body(flat_hbm, bid_hbm, fid_hbm, bdf_hbm, wtf_hbm,
             bd_out, wt_out,
             idx_v, bid_v, fid_v, idx68_m, bd_v, wt_v, sem):
        wid = lax.axis_index("s") * nc + lax.axis_index("c")
        base = wid * rp
        pltpu.sync_copy(flat_hbm.at[pl.ds(base, rp)], idx_v)
        pltpu.sync_copy(bid_hbm.at[pl.ds(base, rp)], bid_v)
        pltpu.sync_copy(fid_hbm.at[pl.ds(base, rp)], fid_v)
        _fill_bd_indices(bid_v, fid_v, idx68_m)
        copies = [pltpu.async_copy(wtf_hbm.at[idx_v], wt_v, sem)]
        for k in range(rows68):
            copies.append(
                pltpu.async_copy(bdf_hbm.at[idx68_m.at[k]], bd_v.at[k], sem))
        for cp in copies:
            cp.wait()
        pltpu.sync_copy(bd_v, bd_out.at[pl.ds(wid * rows68, rows68)])
        pltpu.sync_copy(wt_v, wt_out.at[pl.ds(base, rp)])

    return pl.kernel(
        body,
        out_type=[
            jax.ShapeDtypeStruct((nw * rows68, _CHUNK), jnp.float32),
            jax.ShapeDtypeStruct((_P,), jnp.float32),
        ],
        mesh=plsc.VectorSubcoreMesh(core_axis_name="c",
                                    subcore_axis_name="s"),
        scratch_types=[
            pltpu.VMEM((rp,), jnp.int32),
            pltpu.VMEM((rp,), jnp.int32),
            pltpu.VMEM((rp,), jnp.int32),
            pltpu.VMEM((rows68, _CHUNK), jnp.int32),
            pltpu.VMEM((rows68, _CHUNK), jnp.float32),
            pltpu.VMEM((rp,), jnp.float32),
            pltpu.SemaphoreType.DMA,
        ],
        compiler_params=pltpu.CompilerParams(use_tc_tiling_on_sc=False,
                                             needs_layout_passes=False),
    )


def _dfl_body(bd_ref, lab_ref, wt4_ref, o_ref):
    bd = bd_ref[...]
    labf = lab_ref[...]
    dl = jnp.floor(labf)
    wl = dl + 1.0 - labf
    wr = labf - dl
    dli = dl.astype(jnp.int32)
    dri = jnp.clip(dli + 1, 0, _REGN - 1)
    m = jnp.max(bd, axis=0, keepdims=True)
    lse = m + jnp.log(jnp.sum(jnp.exp(bd - m), axis=0, keepdims=True))
    ii = lax.broadcasted_iota(jnp.int32, bd.shape, 0)
    pick_l = jnp.sum(jnp.where(ii == dli, bd, 0.0), axis=0, keepdims=True)
    pick_r = jnp.sum(jnp.where(ii == dri, bd, 0.0), axis=0, keepdims=True)
    dfl = (lse - pick_l) * wl + (lse - pick_r) * wr
    o_ref[...] = jnp.sum(dfl * wt4_ref[...]).reshape(1, 1)


def _dfl_pass(bd17, lab17, wt4):
    return pl.pallas_call(
        _dfl_body,
        out_shape=jax.ShapeDtypeStruct((1, 1), jnp.float32),
    )(bd17, lab17, wt4)


def kernel(cls_score, predicted_bbox, bbox_distribution, num_positive_anchors,
           batch_ids, feat_ids, class_labels, target_boxes):
    lab = class_labels.astype(jnp.int32).reshape(_N, _H, 1, _W)
    cls_t = jnp.transpose(cls_score, (0, 2, 1, 3))
    pb_t = jnp.transpose(predicted_bbox, (0, 1, 3, 2))
    bd_t = jnp.transpose(bbox_distribution, (1, 3, 0, 2))

    flat_pos = (batch_ids.astype(jnp.int32) * _HW
                + feat_ids.astype(jnp.int32))
    tbd = _sc_scatter_kernel()(flat_pos, target_boxes.reshape(-1))
    tb_t = tbd.reshape(_N, _H, 8, 128)

    s4, wt_dense, bd_pad = _dense_pass(cls_t, lab, pb_t, tb_t, bd_t)

    bdf = bd_pad.reshape(-1)
    bd, wtv = _sc_gather_kernel()(
        flat_pos, batch_ids.astype(jnp.int32), feat_ids.astype(jnp.int32),
        bdf, wt_dense.reshape(_N * _HW))

    bd17 = bd.reshape(4 * _P, _REGN).T
    lab17 = (target_boxes.reshape(1, 4 * _P) * (_REGN - 1.0))
    dflw = _dfl_pass(bd17, lab17, jnp.repeat(wtv, 4).reshape(1, 4 * _P))

    num_pos = jnp.maximum(num_positive_anchors, 1.0)
    qfl_sum = s4[0, 0] + s4[0, 1]
    wt_sum = s4[0, 2]
    loss_qfl = qfl_sum / num_pos
    loss_iou = s4[0, 3] / wt_sum
    loss_dfl = (dflw[0, 0] / 4.0) / wt_sum
    loss = loss_qfl + 2.0 * loss_iou + 0.25 * loss_dfl
    return loss, loss_qfl, loss_iou, loss_dfl
